# baseline (speedup 1.0000x reference)
import functools

import jax
import jax.numpy as jnp
from jax import lax
from jax.experimental import pallas as pl
from jax.experimental.pallas import tpu as pltpu
from jax.experimental.pallas import tpu_sc as plsc

B = 1024
T = 200
V = 100000
D = 64
S = 128
KW = 4


_CHUNK = 128
_NBUF = 10


def _sc_gather(table, idx3d):
    info = plsc.get_sparse_core_info()
    nw = info.num_cores * info.num_subcores
    chunks_per_w = idx3d.shape[1]
    n_rows = nw * chunks_per_w * _CHUNK
    groups = chunks_per_w // _NBUF

    mesh = plsc.VectorSubcoreMesh(core_axis_name="c", subcore_axis_name="s")

    @functools.partial(
        pl.kernel,
        mesh=mesh,
        out_type=jax.ShapeDtypeStruct((n_rows, D), jnp.float32),
        scratch_types=[
            pltpu.VMEM((chunks_per_w, _CHUNK), jnp.int32),
            pltpu.VMEM((_NBUF, _CHUNK, D), jnp.float32),
            pltpu.SemaphoreType.DMA,
            pltpu.SemaphoreType.DMA,
        ],
        compiler_params=pltpu.CompilerParams(use_tc_tiling_on_sc=False),
    )
    def k(table_hbm, idx_hbm, out_hbm, idx_v, rows_v, gsem, osem):
        wid = lax.axis_index("s") * info.num_cores + lax.axis_index("c")
        chunk0 = wid * chunks_per_w
        pltpu.sync_copy(idx_hbm.at[wid], idx_v)

        def group(g, _):
            base = g * _NBUF
            for b in range(_NBUF):
                pltpu.async_copy(
                    table_hbm.at[idx_v.at[base + b]], rows_v.at[b], gsem)
            for b in range(_NBUF):
                pltpu.make_async_copy(
                    table_hbm.at[idx_v.at[base + b]], rows_v.at[b], gsem).wait()
            for b in range(_NBUF):
                row0 = (chunk0 + base + b) * _CHUNK
                pltpu.async_copy(
                    rows_v.at[b], out_hbm.at[pl.ds(row0, _CHUNK)], osem)
            for b in range(_NBUF):
                row0 = (chunk0 + base + b) * _CHUNK
                pltpu.make_async_copy(
                    rows_v.at[b], out_hbm.at[pl.ds(row0, _CHUNK)], osem).wait()
            return ()

        lax.fori_loop(0, groups, group, (), unroll=False)

    return k(table, idx3d)


_BT = 64
_TP = T // 2
_RP = _BT * _TP


def _encoder_body(h_ref, w1_ref, b1_ref, w2_ref, b2_ref, lng_ref, lnb_ref,
                  wg_ref, bg_ref, wr_ref, br_ref, ctx_ref):
    f32 = jnp.float32
    hp = h_ref[...]
    ff = jnp.maximum(
        jnp.dot(hp, w1_ref[...], preferred_element_type=f32) + b1_ref[...],
        0.0)
    ffp = jnp.dot(ff, w2_ref[...], preferred_element_type=f32) + b2_ref[...]
    yp = hp + ffp

    lane = lax.broadcasted_iota(jnp.int32, (_RP, 2 * D), 1)
    m_e = yp[:, :D].mean(axis=-1, keepdims=True)
    m_o = yp[:, D:].mean(axis=-1, keepdims=True)
    m_b = jnp.where(lane < D, m_e, m_o)
    d2 = (yp - m_b) ** 2
    v_e = d2[:, :D].mean(axis=-1, keepdims=True)
    v_o = d2[:, D:].mean(axis=-1, keepdims=True)
    v_b = jnp.where(lane < D, v_e, v_o)
    hidp = (yp - m_b) / jnp.sqrt(v_b + 1e-5) * lng_ref[...] + lnb_ref[...]

    gate = jnp.dot(hidp, wg_ref[...], preferred_element_type=f32) \
        + bg_ref[...]
    s_e = gate[:, :S].mean(axis=-1).reshape(_BT, _TP)
    s_o = gate[:, S:].mean(axis=-1).reshape(_BT, _TP)

    row6k = lax.broadcasted_iota(jnp.int32, (_BT, _RP), 1)
    bat6k = lax.broadcasted_iota(jnp.int32, (_BT, _RP), 0)
    sel_q = (row6k == bat6k * _TP + (_TP - 1)).astype(f32)
    rep_b = bat6k == row6k // _TP
    rowR = lax.broadcasted_iota(jnp.int32, (_RP, _BT), 0)
    batC = lax.broadcasted_iota(jnp.int32, (_RP, _BT), 1)
    rep_bT = (batC == rowR // _TP).astype(f32)

    q = jnp.dot(sel_q, hidp[:, D:], preferred_element_type=f32)
    q = jnp.dot(q, wr_ref[...], preferred_element_type=f32) + br_ref[...]

    qrow = jnp.dot(rep_bT, q, preferred_element_type=f32)
    hpt = hidp.astype(jnp.bfloat16).astype(f32)
    qt = qrow.astype(jnp.bfloat16).astype(f32)
    d_e = (hpt[:, :D] * qt).sum(axis=-1).reshape(_BT, _TP)
    d_o = (hpt[:, D:] * qt).sum(axis=-1).reshape(_BT, _TP)

    col = lax.broadcasted_iota(jnp.int32, (_BT, _TP), 1)
    tpos_e = 2 * col
    tpos_o = 2 * col + 1
    neg = jnp.float32(-1e30)
    work_e = s_e
    work_o = jnp.where(tpos_o >= T - 1, neg, s_o)
    big = jnp.int32(2 * T)
    sel_logit = []
    masks_e = []
    masks_o = []
    for _ in range(KW):
        mx = jnp.maximum(work_e.max(axis=1, keepdims=True),
                         work_o.max(axis=1, keepdims=True))
        cand_e = jnp.where(work_e == mx, tpos_e, big)
        cand_o = jnp.where(work_o == mx, tpos_o, big)
        pick = jnp.minimum(cand_e.min(axis=1, keepdims=True),
                           cand_o.min(axis=1, keepdims=True))
        one_e = tpos_e == pick
        one_o = tpos_o == pick
        masks_e.append(one_e)
        masks_o.append(one_o)
        sel_logit.append(jnp.where(one_e, d_e, 0.0).sum(axis=1)
                         + jnp.where(one_o, d_o, 0.0).sum(axis=1))
        work_e = jnp.where(one_e, neg, work_e)
        work_o = jnp.where(one_o, neg, work_o)

    s = jnp.stack(sel_logit, axis=1)
    mmax = jnp.maximum(s.max(axis=1), 0.0)
    e = jnp.exp(s - mmax[:, None])
    z = e.sum(axis=1) + (S - KW) * jnp.exp(-mmax)
    a = e / z[:, None]

    w_e = jnp.zeros((_BT, _TP), f32)
    w_o = jnp.zeros((_BT, _TP), f32)
    for kk in range(KW):
        w_e = w_e + jnp.where(masks_e[kk], a[:, kk:kk + 1], 0.0)
        w_o = w_o + jnp.where(masks_o[kk], a[:, kk:kk + 1], 0.0)

    jJ = lax.broadcasted_iota(jnp.int32, (_TP, _RP), 0)
    rR = lax.broadcasted_iota(jnp.int32, (_TP, _RP), 1)
    tile = (rR % _TP == jJ).astype(f32)
    hi = jax.lax.Precision.HIGHEST
    m_ew = jnp.where(rep_b, jnp.dot(w_e, tile, precision=hi,
                                    preferred_element_type=f32), 0.0)
    m_ow = jnp.where(rep_b, jnp.dot(w_o, tile, precision=hi,
                                    preferred_element_type=f32), 0.0)
    ctx_e = jnp.dot(m_ew, hidp[:, :D], precision=hi,
                    preferred_element_type=f32)
    ctx_o = jnp.dot(m_ow, hidp[:, D:], precision=hi,
                    preferred_element_type=f32)
    ctx_ref[...] = ctx_e + ctx_o


def _encoder(h2, w1p, b1p, w2p, b2p, lngp, lnbp, wgp, bgp, wr, br):
    grid = B // _BT
    full = lambda shape: pl.BlockSpec(shape, lambda i: (0,) * len(shape))
    return pl.pallas_call(
        _encoder_body,
        grid=(grid,),
        in_specs=[
            pl.BlockSpec((_RP, 2 * D), lambda i: (i, 0)),
            full((2 * D, 4 * D)), full((4 * D,)),
            full((4 * D, 2 * D)), full((2 * D,)),
            full((2 * D,)), full((2 * D,)),
            full((2 * D, 2 * S)), full((2 * S,)),
            full((D, D)), full((D,)),
        ],
        out_specs=pl.BlockSpec((_BT, D), lambda i: (i, 0)),
        out_shape=jax.ShapeDtypeStruct((B, D), jnp.float32),
        compiler_params=pltpu.CompilerParams(
            vmem_limit_bytes=50 * 1024 * 1024),
    )(h2, w1p, b1p, w2p, b2p, lngp, lnbp, wgp, bgp, wr, br)


_VT = 2048


def _proj_body(ctx_ref, wo_ref, bo_ref, out_ref):
    out_ref[...] = jnp.dot(ctx_ref[...].astype(jnp.bfloat16),
                           wo_ref[...].astype(jnp.bfloat16),
                           preferred_element_type=jnp.float32) + bo_ref[...]


def _projection(ctx, wo, bo2d):
    grid = pl.cdiv(V, _VT)
    return pl.pallas_call(
        _proj_body,
        grid=(grid,),
        in_specs=[
            pl.BlockSpec((B, D), lambda j: (0, 0)),
            pl.BlockSpec((D, _VT), lambda j: (0, j)),
            pl.BlockSpec((1, _VT), lambda j: (0, j)),
        ],
        out_specs=pl.BlockSpec((B, _VT), lambda j: (0, j)),
        out_shape=jax.ShapeDtypeStruct((B, V), jnp.float32),
    )(ctx, wo, bo2d)


def _blockdiag(w):
    z = jnp.zeros_like(w)
    return jnp.concatenate(
        [jnp.concatenate([w, z], axis=1), jnp.concatenate([z, w], axis=1)],
        axis=0)


def kernel(seq, table, w1, b1, w2, b2, ln_g, ln_b, wg, bg, wr, br, wo, bo):
    idx3d = seq.reshape(32, -1, _CHUNK).astype(jnp.int32)
    h = _sc_gather(table, idx3d)
    h2 = h.reshape(B * T // 2, 2 * D)
    w1p = _blockdiag(w1)
    w2p = _blockdiag(w2)
    wgp = _blockdiag(wg)
    two = lambda v: jnp.concatenate([v, v])
    ctx = _encoder(h2, w1p, two(b1), w2p, two(b2), two(ln_g), two(ln_b),
                   wgp, two(bg), wr, br)
    return _projection(ctx, wo, bo.reshape(1, V))


# speedup vs baseline: 1.2143x; 1.2143x over previous
import functools

import jax
import jax.numpy as jnp
from jax import lax
from jax.experimental import pallas as pl
from jax.experimental.pallas import tpu as pltpu
from jax.experimental.pallas import tpu_sc as plsc

B = 1024
T = 200
V = 100000
D = 64
S = 128
KW = 4


_CHUNK = 128
_NBUF = 10


def _sc_gather(table, idx3d):
    info = plsc.get_sparse_core_info()
    nw = info.num_cores * info.num_subcores
    chunks_per_w = idx3d.shape[1]
    n_rows = nw * chunks_per_w * _CHUNK
    groups = chunks_per_w // _NBUF

    mesh = plsc.VectorSubcoreMesh(core_axis_name="c", subcore_axis_name="s")

    @functools.partial(
        pl.kernel,
        mesh=mesh,
        out_type=jax.ShapeDtypeStruct((n_rows, D), jnp.float32),
        scratch_types=[
            pltpu.VMEM((chunks_per_w, _CHUNK), jnp.int32),
            pltpu.VMEM((_NBUF, _CHUNK, D), jnp.float32),
            pltpu.SemaphoreType.DMA,
            pltpu.SemaphoreType.DMA,
        ],
        compiler_params=pltpu.CompilerParams(use_tc_tiling_on_sc=False),
    )
    def k(table_hbm, idx_hbm, out_hbm, idx_v, rows_v, gsem, osem):
        wid = lax.axis_index("s") * info.num_cores + lax.axis_index("c")
        chunk0 = wid * chunks_per_w
        pltpu.sync_copy(idx_hbm.at[wid], idx_v)

        def group(g, _):
            base = g * _NBUF
            for b in range(_NBUF):
                pltpu.async_copy(
                    table_hbm.at[idx_v.at[base + b]], rows_v.at[b], gsem)
            for b in range(_NBUF):
                pltpu.make_async_copy(
                    table_hbm.at[idx_v.at[base + b]], rows_v.at[b], gsem).wait()
            for b in range(_NBUF):
                row0 = (chunk0 + base + b) * _CHUNK
                pltpu.async_copy(
                    rows_v.at[b], out_hbm.at[pl.ds(row0, _CHUNK)], osem)
            for b in range(_NBUF):
                row0 = (chunk0 + base + b) * _CHUNK
                pltpu.make_async_copy(
                    rows_v.at[b], out_hbm.at[pl.ds(row0, _CHUNK)], osem).wait()
            return ()

        lax.fori_loop(0, groups, group, (), unroll=False)

    return k(table, idx3d)


_BT = 64
_TP = T // 2
_RP = _BT * _TP


def _encoder_body(h_ref, w1_ref, b1_ref, w2_ref, b2_ref, lng_ref, lnb_ref,
                  wg_ref, bg_ref, wr_ref, br_ref, ctx_ref):
    f32 = jnp.float32
    hp = h_ref[...]
    ff = jnp.maximum(
        jnp.dot(hp, w1_ref[...], preferred_element_type=f32) + b1_ref[...],
        0.0)
    ffp = jnp.dot(ff, w2_ref[...], preferred_element_type=f32) + b2_ref[...]
    yp = hp + ffp

    lane = lax.broadcasted_iota(jnp.int32, (_RP, 2 * D), 1)
    m_e = yp[:, :D].mean(axis=-1, keepdims=True)
    m_o = yp[:, D:].mean(axis=-1, keepdims=True)
    m_b = jnp.where(lane < D, m_e, m_o)
    d2 = (yp - m_b) ** 2
    v_e = d2[:, :D].mean(axis=-1, keepdims=True)
    v_o = d2[:, D:].mean(axis=-1, keepdims=True)
    v_b = jnp.where(lane < D, v_e, v_o)
    hidp = (yp - m_b) / jnp.sqrt(v_b + 1e-5) * lng_ref[...] + lnb_ref[...]

    gate = jnp.dot(hidp, wg_ref[...], preferred_element_type=f32) \
        + bg_ref[...]
    s_e = gate[:, :S].mean(axis=-1).reshape(_BT, _TP)
    s_o = gate[:, S:].mean(axis=-1).reshape(_BT, _TP)

    hid3 = hidp.reshape(_BT, _TP, 2 * D)

    q = jnp.dot(hid3[:, _TP - 1, D:], wr_ref[...],
                preferred_element_type=f32) + br_ref[...]

    hpt3 = hid3.astype(jnp.bfloat16).astype(f32)
    qt = q.astype(jnp.bfloat16).astype(f32)[:, None, :]
    d_e = (hpt3[:, :, :D] * qt).sum(axis=-1)
    d_o = (hpt3[:, :, D:] * qt).sum(axis=-1)

    col = lax.broadcasted_iota(jnp.int32, (_BT, _TP), 1)
    tpos_e = 2 * col
    tpos_o = 2 * col + 1
    neg = jnp.float32(-1e30)
    work_e = s_e
    work_o = jnp.where(tpos_o >= T - 1, neg, s_o)
    big = jnp.int32(2 * T)
    sel_logit = []
    masks_e = []
    masks_o = []
    for _ in range(KW):
        mx = jnp.maximum(work_e.max(axis=1, keepdims=True),
                         work_o.max(axis=1, keepdims=True))
        cand_e = jnp.where(work_e == mx, tpos_e, big)
        cand_o = jnp.where(work_o == mx, tpos_o, big)
        pick = jnp.minimum(cand_e.min(axis=1, keepdims=True),
                           cand_o.min(axis=1, keepdims=True))
        one_e = tpos_e == pick
        one_o = tpos_o == pick
        masks_e.append(one_e)
        masks_o.append(one_o)
        sel_logit.append(jnp.where(one_e, d_e, 0.0).sum(axis=1)
                         + jnp.where(one_o, d_o, 0.0).sum(axis=1))
        work_e = jnp.where(one_e, neg, work_e)
        work_o = jnp.where(one_o, neg, work_o)

    s = jnp.stack(sel_logit, axis=1)
    mmax = jnp.maximum(s.max(axis=1), 0.0)
    e = jnp.exp(s - mmax[:, None])
    z = e.sum(axis=1) + (S - KW) * jnp.exp(-mmax)
    a = e / z[:, None]

    w_e = jnp.zeros((_BT, _TP), f32)
    w_o = jnp.zeros((_BT, _TP), f32)
    for kk in range(KW):
        w_e = w_e + jnp.where(masks_e[kk], a[:, kk:kk + 1], 0.0)
        w_o = w_o + jnp.where(masks_o[kk], a[:, kk:kk + 1], 0.0)

    lane3 = lax.broadcasted_iota(jnp.int32, (_BT, _TP, 2 * D), 2)
    w3 = jnp.where(lane3 < D, w_e[:, :, None], w_o[:, :, None])
    ctx128 = (hid3 * w3).sum(axis=1)
    ctx_ref[...] = ctx128[:, :D] + ctx128[:, D:]


def _encoder(h2, w1p, b1p, w2p, b2p, lngp, lnbp, wgp, bgp, wr, br):
    grid = B // _BT
    full = lambda shape: pl.BlockSpec(shape, lambda i: (0,) * len(shape))
    return pl.pallas_call(
        _encoder_body,
        grid=(grid,),
        in_specs=[
            pl.BlockSpec((_RP, 2 * D), lambda i: (i, 0)),
            full((2 * D, 4 * D)), full((4 * D,)),
            full((4 * D, 2 * D)), full((2 * D,)),
            full((2 * D,)), full((2 * D,)),
            full((2 * D, 2 * S)), full((2 * S,)),
            full((D, D)), full((D,)),
        ],
        out_specs=pl.BlockSpec((_BT, D), lambda i: (i, 0)),
        out_shape=jax.ShapeDtypeStruct((B, D), jnp.float32),
        compiler_params=pltpu.CompilerParams(
            vmem_limit_bytes=50 * 1024 * 1024),
    )(h2, w1p, b1p, w2p, b2p, lngp, lnbp, wgp, bgp, wr, br)


_VT = 2048


def _proj_body(ctx_ref, wo_ref, bo_ref, out_ref):
    out_ref[...] = jnp.dot(ctx_ref[...].astype(jnp.bfloat16),
                           wo_ref[...].astype(jnp.bfloat16),
                           preferred_element_type=jnp.float32) + bo_ref[...]


def _projection(ctx, wo, bo2d):
    grid = pl.cdiv(V, _VT)
    return pl.pallas_call(
        _proj_body,
        grid=(grid,),
        in_specs=[
            pl.BlockSpec((B, D), lambda j: (0, 0)),
            pl.BlockSpec((D, _VT), lambda j: (0, j)),
            pl.BlockSpec((1, _VT), lambda j: (0, j)),
        ],
        out_specs=pl.BlockSpec((B, _VT), lambda j: (0, j)),
        out_shape=jax.ShapeDtypeStruct((B, V), jnp.float32),
    )(ctx, wo, bo2d)


def _blockdiag(w):
    z = jnp.zeros_like(w)
    return jnp.concatenate(
        [jnp.concatenate([w, z], axis=1), jnp.concatenate([z, w], axis=1)],
        axis=0)


def kernel(seq, table, w1, b1, w2, b2, ln_g, ln_b, wg, bg, wr, br, wo, bo):
    idx3d = seq.reshape(32, -1, _CHUNK).astype(jnp.int32)
    h = _sc_gather(table, idx3d)
    h2 = h.reshape(B * T // 2, 2 * D)
    w1p = _blockdiag(w1)
    w2p = _blockdiag(w2)
    wgp = _blockdiag(wg)
    two = lambda v: jnp.concatenate([v, v])
    ctx = _encoder(h2, w1p, two(b1), w2p, two(b2), two(ln_g), two(ln_b),
                   wgp, two(bg), wr, br)
    return _projection(ctx, wo, bo.reshape(1, V))


# speedup vs baseline: 1.2575x; 1.0355x over previous
import functools

import jax
import jax.numpy as jnp
from jax import lax
from jax.experimental import pallas as pl
from jax.experimental.pallas import tpu as pltpu
from jax.experimental.pallas import tpu_sc as plsc

B = 1024
T = 200
V = 100000
D = 64
S = 128
KW = 4


_CHUNK = 128
_NBUF = 10


def _sc_gather(table, idx3d):
    info = plsc.get_sparse_core_info()
    nw = info.num_cores * info.num_subcores
    chunks_per_w = idx3d.shape[1]
    n_rows = nw * chunks_per_w * _CHUNK
    groups = chunks_per_w // _NBUF

    mesh = plsc.VectorSubcoreMesh(core_axis_name="c", subcore_axis_name="s")

    @functools.partial(
        pl.kernel,
        mesh=mesh,
        out_type=jax.ShapeDtypeStruct((n_rows, D), jnp.float32),
        scratch_types=[
            pltpu.VMEM((chunks_per_w, _CHUNK), jnp.int32),
            pltpu.VMEM((_NBUF, _CHUNK, D), jnp.float32),
            pltpu.SemaphoreType.DMA,
            pltpu.SemaphoreType.DMA,
        ],
        compiler_params=pltpu.CompilerParams(use_tc_tiling_on_sc=False),
    )
    def k(table_hbm, idx_hbm, out_hbm, idx_v, rows_v, gsem, osem):
        wid = lax.axis_index("s") * info.num_cores + lax.axis_index("c")
        chunk0 = wid * chunks_per_w
        pltpu.sync_copy(idx_hbm.at[wid], idx_v)

        def group(g, _):
            base = g * _NBUF
            for b in range(_NBUF):
                pltpu.async_copy(
                    table_hbm.at[idx_v.at[base + b]], rows_v.at[b], gsem)
            for b in range(_NBUF):
                pltpu.make_async_copy(
                    table_hbm.at[idx_v.at[base + b]], rows_v.at[b], gsem).wait()
            for b in range(_NBUF):
                row0 = (chunk0 + base + b) * _CHUNK
                pltpu.async_copy(
                    rows_v.at[b], out_hbm.at[pl.ds(row0, _CHUNK)], osem)
            for b in range(_NBUF):
                row0 = (chunk0 + base + b) * _CHUNK
                pltpu.make_async_copy(
                    rows_v.at[b], out_hbm.at[pl.ds(row0, _CHUNK)], osem).wait()
            return ()

        lax.fori_loop(0, groups, group, (), unroll=False)

    return k(table, idx3d)


_BT = 64
_TP = T // 2
_RP = _BT * _TP


def _encoder_body(h_ref, w1_ref, b1_ref, w2_ref, b2_ref, lng_ref, lnb_ref,
                  wg_ref, bg_ref, wr_ref, br_ref, ctx_ref):
    f32 = jnp.float32
    hp = h_ref[...]
    ff = jnp.maximum(
        jnp.dot(hp, w1_ref[...], preferred_element_type=f32) + b1_ref[...],
        0.0)
    ffp = jnp.dot(ff, w2_ref[...], preferred_element_type=f32) + b2_ref[...]
    yp = hp + ffp

    lane = lax.broadcasted_iota(jnp.int32, (_RP, 2 * D), 1)
    m_e = yp[:, :D].mean(axis=-1, keepdims=True)
    m_o = yp[:, D:].mean(axis=-1, keepdims=True)
    m_b = jnp.where(lane < D, m_e, m_o)
    d2 = (yp - m_b) ** 2
    v_e = d2[:, :D].mean(axis=-1, keepdims=True)
    v_o = d2[:, D:].mean(axis=-1, keepdims=True)
    v_b = jnp.where(lane < D, v_e, v_o)
    hidp = (yp - m_b) / jnp.sqrt(v_b + 1e-5) * lng_ref[...] + lnb_ref[...]

    gate = jnp.dot(hidp, wg_ref[...], preferred_element_type=f32) \
        + bg_ref[...]
    G = _BT // 2
    R2 = 2 * _TP
    s_e = gate[:, :S].mean(axis=-1).reshape(G, R2)
    s_o = gate[:, S:].mean(axis=-1).reshape(G, R2)

    hid3 = hidp.reshape(G, R2, 2 * D)

    qab = jnp.concatenate([hid3[:, _TP - 1, D:], hid3[:, R2 - 1, D:]],
                          axis=0)
    q2 = jnp.dot(qab, wr_ref[...], preferred_element_type=f32) + br_ref[...]

    hpt3 = hid3.astype(jnp.bfloat16).astype(f32)
    q2t = q2.astype(jnp.bfloat16).astype(f32)
    rr3 = lax.broadcasted_iota(jnp.int32, (G, R2, 1), 1)
    qt3 = jnp.where(rr3 < _TP, q2t[:G][:, None, :], q2t[G:][:, None, :])
    d_e = (hpt3[:, :, :D] * qt3).sum(axis=-1)
    d_o = (hpt3[:, :, D:] * qt3).sum(axis=-1)

    col = lax.broadcasted_iota(jnp.int32, (G, R2), 1)
    first = col < _TP
    tpos_e = 2 * (col % _TP)
    tpos_o = tpos_e + 1
    neg = jnp.float32(-1e30)
    big = jnp.int32(2 * T)
    work_e = s_e
    work_o = jnp.where(tpos_o >= T - 1, neg, s_o)
    logits1 = []
    logits2 = []
    masks_e = []
    masks_o = []
    for _ in range(KW):
        me1 = jnp.where(first, work_e, neg).max(axis=1, keepdims=True)
        mo1 = jnp.where(first, work_o, neg).max(axis=1, keepdims=True)
        me2 = jnp.where(first, neg, work_e).max(axis=1, keepdims=True)
        mo2 = jnp.where(first, neg, work_o).max(axis=1, keepdims=True)
        mx = jnp.where(first, jnp.maximum(me1, mo1), jnp.maximum(me2, mo2))
        cand_e = jnp.where(work_e == mx, tpos_e, big)
        cand_o = jnp.where(work_o == mx, tpos_o, big)
        cand = jnp.minimum(cand_e, cand_o)
        p1 = jnp.where(first, cand, big).min(axis=1, keepdims=True)
        p2 = jnp.where(first, big, cand).min(axis=1, keepdims=True)
        pick = jnp.where(first, p1, p2)
        one_e = tpos_e == pick
        one_o = tpos_o == pick
        masks_e.append(one_e)
        masks_o.append(one_o)
        dsel = jnp.where(one_e, d_e, 0.0) + jnp.where(one_o, d_o, 0.0)
        logits1.append(jnp.where(first, dsel, 0.0).sum(axis=1))
        logits2.append(jnp.where(first, 0.0, dsel).sum(axis=1))
        work_e = jnp.where(one_e, neg, work_e)
        work_o = jnp.where(one_o, neg, work_o)

    def _attn(logit_list):
        s = jnp.stack(logit_list, axis=1)
        mmax = jnp.maximum(s.max(axis=1), 0.0)
        e = jnp.exp(s - mmax[:, None])
        z = e.sum(axis=1) + (S - KW) * jnp.exp(-mmax)
        return e / z[:, None]

    a1 = _attn(logits1)
    a2 = _attn(logits2)

    w_e = jnp.zeros((G, R2), f32)
    w_o = jnp.zeros((G, R2), f32)
    for kk in range(KW):
        a_k = jnp.where(first, a1[:, kk:kk + 1], a2[:, kk:kk + 1])
        w_e = w_e + jnp.where(masks_e[kk], a_k, 0.0)
        w_o = w_o + jnp.where(masks_o[kk], a_k, 0.0)

    lane3 = lax.broadcasted_iota(jnp.int32, (G, R2, 2 * D), 2)
    w3 = jnp.where(lane3 < D, w_e[:, :, None], w_o[:, :, None])
    acc = hid3 * w3
    first3 = rr3 < _TP
    ctx_a = jnp.where(first3, acc, 0.0).sum(axis=1)
    ctx_b = jnp.where(first3, 0.0, acc).sum(axis=1)
    ctx128 = jnp.stack([ctx_a, ctx_b], axis=1).reshape(_BT, 2 * D)
    ctx_ref[...] = ctx128[:, :D] + ctx128[:, D:]


def _encoder(h2, w1p, b1p, w2p, b2p, lngp, lnbp, wgp, bgp, wr, br):
    grid = B // _BT
    full = lambda shape: pl.BlockSpec(shape, lambda i: (0,) * len(shape))
    return pl.pallas_call(
        _encoder_body,
        grid=(grid,),
        in_specs=[
            pl.BlockSpec((_RP, 2 * D), lambda i: (i, 0)),
            full((2 * D, 4 * D)), full((4 * D,)),
            full((4 * D, 2 * D)), full((2 * D,)),
            full((2 * D,)), full((2 * D,)),
            full((2 * D, 2 * S)), full((2 * S,)),
            full((D, D)), full((D,)),
        ],
        out_specs=pl.BlockSpec((_BT, D), lambda i: (i, 0)),
        out_shape=jax.ShapeDtypeStruct((B, D), jnp.float32),
        compiler_params=pltpu.CompilerParams(
            vmem_limit_bytes=50 * 1024 * 1024),
    )(h2, w1p, b1p, w2p, b2p, lngp, lnbp, wgp, bgp, wr, br)


_VT = 2048


def _proj_body(ctx_ref, wo_ref, bo_ref, out_ref):
    out_ref[...] = jnp.dot(ctx_ref[...].astype(jnp.bfloat16),
                           wo_ref[...].astype(jnp.bfloat16),
                           preferred_element_type=jnp.float32) + bo_ref[...]


def _projection(ctx, wo, bo2d):
    grid = pl.cdiv(V, _VT)
    return pl.pallas_call(
        _proj_body,
        grid=(grid,),
        in_specs=[
            pl.BlockSpec((B, D), lambda j: (0, 0)),
            pl.BlockSpec((D, _VT), lambda j: (0, j)),
            pl.BlockSpec((1, _VT), lambda j: (0, j)),
        ],
        out_specs=pl.BlockSpec((B, _VT), lambda j: (0, j)),
        out_shape=jax.ShapeDtypeStruct((B, V), jnp.float32),
    )(ctx, wo, bo2d)


def _blockdiag(w):
    z = jnp.zeros_like(w)
    return jnp.concatenate(
        [jnp.concatenate([w, z], axis=1), jnp.concatenate([z, w], axis=1)],
        axis=0)


def kernel(seq, table, w1, b1, w2, b2, ln_g, ln_b, wg, bg, wr, br, wo, bo):
    idx3d = seq.reshape(32, -1, _CHUNK).astype(jnp.int32)
    h = _sc_gather(table, idx3d)
    h2 = h.reshape(B * T // 2, 2 * D)
    w1p = _blockdiag(w1)
    w2p = _blockdiag(w2)
    wgp = _blockdiag(wg)
    two = lambda v: jnp.concatenate([v, v])
    ctx = _encoder(h2, w1p, two(b1), w2p, two(b2), two(ln_g), two(ln_b),
                   wgp, two(bg), wr, br)
    return _projection(ctx, wo, bo.reshape(1, V))


# speedup vs baseline: 1.2610x; 1.0028x over previous
import functools

import jax
import jax.numpy as jnp
from jax import lax
from jax.experimental import pallas as pl
from jax.experimental.pallas import tpu as pltpu
from jax.experimental.pallas import tpu_sc as plsc

B = 1024
T = 200
V = 100000
D = 64
S = 128
KW = 4


_CHUNK = 128
_NBUF = 10


def _sc_gather(table, idx3d):
    info = plsc.get_sparse_core_info()
    nw = info.num_cores * info.num_subcores
    chunks_per_w = idx3d.shape[1]
    n_rows = nw * chunks_per_w * _CHUNK
    groups = chunks_per_w // _NBUF

    mesh = plsc.VectorSubcoreMesh(core_axis_name="c", subcore_axis_name="s")

    @functools.partial(
        pl.kernel,
        mesh=mesh,
        out_type=jax.ShapeDtypeStruct((n_rows, D), jnp.float32),
        scratch_types=[
            pltpu.VMEM((chunks_per_w, _CHUNK), jnp.int32),
            pltpu.VMEM((_NBUF, _CHUNK, D), jnp.float32),
            pltpu.SemaphoreType.DMA,
            pltpu.SemaphoreType.DMA,
        ],
        compiler_params=pltpu.CompilerParams(use_tc_tiling_on_sc=False),
    )
    def k(table_hbm, idx_hbm, out_hbm, idx_v, rows_v, gsem, osem):
        wid = lax.axis_index("s") * info.num_cores + lax.axis_index("c")
        chunk0 = wid * chunks_per_w
        pltpu.sync_copy(idx_hbm.at[wid], idx_v)

        def group(g, _):
            base = g * _NBUF
            for b in range(_NBUF):
                pltpu.async_copy(
                    table_hbm.at[idx_v.at[base + b]], rows_v.at[b], gsem)
            for b in range(_NBUF):
                pltpu.make_async_copy(
                    table_hbm.at[idx_v.at[base + b]], rows_v.at[b], gsem).wait()
            for b in range(_NBUF):
                row0 = (chunk0 + base + b) * _CHUNK
                pltpu.async_copy(
                    rows_v.at[b], out_hbm.at[pl.ds(row0, _CHUNK)], osem)
            for b in range(_NBUF):
                row0 = (chunk0 + base + b) * _CHUNK
                pltpu.make_async_copy(
                    rows_v.at[b], out_hbm.at[pl.ds(row0, _CHUNK)], osem).wait()
            return ()

        lax.fori_loop(0, groups, group, (), unroll=False)

    return k(table, idx3d)


_BT = 64
_TP = T // 2
_RP = _BT * _TP


def _encoder_body(h_ref, w1_ref, b1_ref, w2_ref, b2_ref, lng_ref, lnb_ref,
                  wg_ref, bg_ref, wr_ref, br_ref, ctx_ref):
    f32 = jnp.float32
    hp = h_ref[...]
    ff = jnp.maximum(
        jnp.dot(hp, w1_ref[...], preferred_element_type=f32) + b1_ref[...],
        0.0)
    ffp = jnp.dot(ff, w2_ref[...], preferred_element_type=f32) + b2_ref[...]
    yp = hp + ffp

    lane = lax.broadcasted_iota(jnp.int32, (_RP, 2 * D), 1)
    m_e = yp[:, :D].mean(axis=-1, keepdims=True)
    m_o = yp[:, D:].mean(axis=-1, keepdims=True)
    m_b = jnp.where(lane < D, m_e, m_o)
    d2 = (yp - m_b) ** 2
    v_e = d2[:, :D].mean(axis=-1, keepdims=True)
    v_o = d2[:, D:].mean(axis=-1, keepdims=True)
    v_b = jnp.where(lane < D, v_e, v_o)
    hidp = (yp - m_b) / jnp.sqrt(v_b + 1e-5) * lng_ref[...] + lnb_ref[...]

    gate = jnp.dot(hidp, wg_ref[...], preferred_element_type=f32) \
        + bg_ref[...]
    G = _BT // 2
    R2 = 2 * _TP
    s_e = gate[:, :S].mean(axis=-1).reshape(G, R2)
    s_o = gate[:, S:].mean(axis=-1).reshape(G, R2)

    hid3 = hidp.reshape(G, R2, 2 * D)

    qab = jnp.concatenate([hid3[:, _TP - 1, D:], hid3[:, R2 - 1, D:]],
                          axis=0)
    q2 = jnp.dot(qab, wr_ref[...], preferred_element_type=f32) + br_ref[...]

    hpt3 = hid3.astype(jnp.bfloat16).astype(f32)
    q2t = q2.astype(jnp.bfloat16).astype(f32)
    rr3 = lax.broadcasted_iota(jnp.int32, (G, R2, 1), 1)
    qt3 = jnp.where(rr3 < _TP, q2t[:G][:, None, :], q2t[G:][:, None, :])
    d_e = (hpt3[:, :, :D] * qt3).sum(axis=-1)
    d_o = (hpt3[:, :, D:] * qt3).sum(axis=-1)

    col = lax.broadcasted_iota(jnp.int32, (G, R2), 1)
    first = col < _TP
    tpos_e = 2 * (col % _TP)
    tpos_o = tpos_e + 1
    neg = jnp.float32(-1e30)
    big = jnp.int32(2 * T)
    work_e = s_e
    work_o = jnp.where(tpos_o >= T - 1, neg, s_o)
    logits1 = []
    logits2 = []
    masks_e = []
    masks_o = []
    for _ in range(KW):
        me1 = jnp.where(first, work_e, neg).max(axis=1, keepdims=True)
        mo1 = jnp.where(first, work_o, neg).max(axis=1, keepdims=True)
        me2 = jnp.where(first, neg, work_e).max(axis=1, keepdims=True)
        mo2 = jnp.where(first, neg, work_o).max(axis=1, keepdims=True)
        mx = jnp.where(first, jnp.maximum(me1, mo1), jnp.maximum(me2, mo2))
        cand_e = jnp.where(work_e == mx, tpos_e, big)
        cand_o = jnp.where(work_o == mx, tpos_o, big)
        cand = jnp.minimum(cand_e, cand_o)
        p1 = jnp.where(first, cand, big).min(axis=1, keepdims=True)
        p2 = jnp.where(first, big, cand).min(axis=1, keepdims=True)
        pick = jnp.where(first, p1, p2)
        one_e = tpos_e == pick
        one_o = tpos_o == pick
        masks_e.append(one_e)
        masks_o.append(one_o)
        dsel = jnp.where(one_e, d_e, 0.0) + jnp.where(one_o, d_o, 0.0)
        logits1.append(jnp.where(first, dsel, 0.0).sum(axis=1))
        logits2.append(jnp.where(first, 0.0, dsel).sum(axis=1))
        work_e = jnp.where(one_e, neg, work_e)
        work_o = jnp.where(one_o, neg, work_o)

    def _attn(logit_list):
        s = jnp.stack(logit_list, axis=1)
        mmax = jnp.maximum(s.max(axis=1), 0.0)
        e = jnp.exp(s - mmax[:, None])
        z = e.sum(axis=1) + (S - KW) * jnp.exp(-mmax)
        return e / z[:, None]

    a1 = _attn(logits1)
    a2 = _attn(logits2)

    w_e = jnp.zeros((G, R2), f32)
    w_o = jnp.zeros((G, R2), f32)
    for kk in range(KW):
        a_k = jnp.where(first, a1[:, kk:kk + 1], a2[:, kk:kk + 1])
        w_e = w_e + jnp.where(masks_e[kk], a_k, 0.0)
        w_o = w_o + jnp.where(masks_o[kk], a_k, 0.0)

    lane3 = lax.broadcasted_iota(jnp.int32, (G, R2, 2 * D), 2)
    w3 = jnp.where(lane3 < D, w_e[:, :, None], w_o[:, :, None])
    acc = hid3 * w3
    first3 = rr3 < _TP
    ctx_a = jnp.where(first3, acc, 0.0).sum(axis=1)
    ctx_b = jnp.where(first3, 0.0, acc).sum(axis=1)
    ctx128 = jnp.stack([ctx_a, ctx_b], axis=1).reshape(_BT, 2 * D)
    ctx_ref[...] = ctx128[:, :D] + ctx128[:, D:]


def _encoder(h2, w1p, b1p, w2p, b2p, lngp, lnbp, wgp, bgp, wr, br):
    grid = B // _BT
    full = lambda shape: pl.BlockSpec(shape, lambda i: (0,) * len(shape))
    return pl.pallas_call(
        _encoder_body,
        grid=(grid,),
        in_specs=[
            pl.BlockSpec((_RP, 2 * D), lambda i: (i, 0)),
            full((2 * D, 4 * D)), full((4 * D,)),
            full((4 * D, 2 * D)), full((2 * D,)),
            full((2 * D,)), full((2 * D,)),
            full((2 * D, 2 * S)), full((2 * S,)),
            full((D, D)), full((D,)),
        ],
        out_specs=pl.BlockSpec((_BT, D), lambda i: (i, 0)),
        out_shape=jax.ShapeDtypeStruct((B, D), jnp.float32),
        compiler_params=pltpu.CompilerParams(
            vmem_limit_bytes=50 * 1024 * 1024),
    )(h2, w1p, b1p, w2p, b2p, lngp, lnbp, wgp, bgp, wr, br)


_VT = 4096


def _proj_body(ctx_ref, wo_ref, bo_ref, out_ref):
    out_ref[...] = jnp.dot(ctx_ref[...].astype(jnp.bfloat16),
                           wo_ref[...].astype(jnp.bfloat16),
                           preferred_element_type=jnp.float32) + bo_ref[...]


def _projection(ctx, wo, bo2d):
    grid = pl.cdiv(V, _VT)
    return pl.pallas_call(
        _proj_body,
        grid=(grid,),
        in_specs=[
            pl.BlockSpec((B, D), lambda j: (0, 0)),
            pl.BlockSpec((D, _VT), lambda j: (0, j)),
            pl.BlockSpec((1, _VT), lambda j: (0, j)),
        ],
        out_specs=pl.BlockSpec((B, _VT), lambda j: (0, j)),
        out_shape=jax.ShapeDtypeStruct((B, V), jnp.float32),
        compiler_params=pltpu.CompilerParams(
            vmem_limit_bytes=50 * 1024 * 1024),
    )(ctx, wo, bo2d)


def _blockdiag(w):
    z = jnp.zeros_like(w)
    return jnp.concatenate(
        [jnp.concatenate([w, z], axis=1), jnp.concatenate([z, w], axis=1)],
        axis=0)


def kernel(seq, table, w1, b1, w2, b2, ln_g, ln_b, wg, bg, wr, br, wo, bo):
    idx3d = seq.reshape(32, -1, _CHUNK).astype(jnp.int32)
    h = _sc_gather(table, idx3d)
    h2 = h.reshape(B * T // 2, 2 * D)
    w1p = _blockdiag(w1)
    w2p = _blockdiag(w2)
    wgp = _blockdiag(wg)
    two = lambda v: jnp.concatenate([v, v])
    ctx = _encoder(h2, w1p, two(b1), w2p, two(b2), two(ln_g), two(ln_b),
                   wgp, two(bg), wr, br)
    return _projection(ctx, wo, bo.reshape(1, V))


# speedup vs baseline: 1.4865x; 1.1789x over previous
import functools

import jax
import jax.numpy as jnp
from jax import lax
from jax.experimental import pallas as pl
from jax.experimental.pallas import tpu as pltpu
from jax.experimental.pallas import tpu_sc as plsc

B = 1024
T = 200
V = 100000
D = 64
S = 128
KW = 4


_CHUNK = 128
_NBUF = 10


def _sc_gather(table, idx3d):
    info = plsc.get_sparse_core_info()
    nw = info.num_cores * info.num_subcores
    chunks_per_w = idx3d.shape[1]
    n_rows = nw * chunks_per_w * _CHUNK
    groups = chunks_per_w // _NBUF

    mesh = plsc.VectorSubcoreMesh(core_axis_name="c", subcore_axis_name="s")

    @functools.partial(
        pl.kernel,
        mesh=mesh,
        out_type=jax.ShapeDtypeStruct((n_rows, D), jnp.float32),
        scratch_types=[
            pltpu.VMEM((chunks_per_w, _CHUNK), jnp.int32),
            pltpu.VMEM((_NBUF, _CHUNK, D), jnp.float32),
            pltpu.SemaphoreType.DMA,
            pltpu.SemaphoreType.DMA,
        ],
        compiler_params=pltpu.CompilerParams(use_tc_tiling_on_sc=False),
    )
    def k(table_hbm, idx_hbm, out_hbm, idx_v, rows_v, gsem, osem):
        wid = lax.axis_index("s") * info.num_cores + lax.axis_index("c")
        chunk0 = wid * chunks_per_w
        pltpu.sync_copy(idx_hbm.at[wid], idx_v)

        def group(g, _):
            base = g * _NBUF
            for b in range(_NBUF):
                pltpu.async_copy(
                    table_hbm.at[idx_v.at[base + b]], rows_v.at[b], gsem)
            for b in range(_NBUF):
                pltpu.make_async_copy(
                    table_hbm.at[idx_v.at[base + b]], rows_v.at[b], gsem).wait()
            for b in range(_NBUF):
                row0 = (chunk0 + base + b) * _CHUNK
                pltpu.async_copy(
                    rows_v.at[b], out_hbm.at[pl.ds(row0, _CHUNK)], osem)
            for b in range(_NBUF):
                row0 = (chunk0 + base + b) * _CHUNK
                pltpu.make_async_copy(
                    rows_v.at[b], out_hbm.at[pl.ds(row0, _CHUNK)], osem).wait()
            return ()

        lax.fori_loop(0, groups, group, (), unroll=False)

    return k(table, idx3d)


_BT = 64


def _encoder_body(h_ref, w1_ref, b1_ref, w2_ref, b2_ref, lng_ref, lnb_ref,
                  wg_ref, bg_ref, wr_ref, br_ref, ctx_ref):
    h = h_ref[...]
    x = h.reshape(_BT * T, D)
    ff = jnp.maximum(
        jnp.dot(x, w1_ref[...], preferred_element_type=jnp.float32)
        + b1_ref[...], 0.0)
    ff = jnp.dot(ff, w2_ref[...], preferred_element_type=jnp.float32) \
        + b2_ref[...]
    y = x + ff
    m = y.mean(axis=-1, keepdims=True)
    v = ((y - m) ** 2).mean(axis=-1, keepdims=True)
    hid = (y - m) / jnp.sqrt(v + 1e-5) * lng_ref[...] + lnb_ref[...]

    gate = jnp.dot(hid, wg_ref[...], preferred_element_type=jnp.float32) \
        + bg_ref[...]
    scores = gate.mean(axis=-1).reshape(_BT, T)
    tpos = lax.broadcasted_iota(jnp.int32, (_BT, T), 1)
    neg = jnp.float32(-1e30)
    scores = jnp.where(tpos >= T - 1, neg, scores)

    hid3 = hid.reshape(_BT, T, D)
    q = jnp.dot(hid3[:, T - 1, :], wr_ref[...],
                preferred_element_type=jnp.float32) + br_ref[...]
    hid3t = hid3.astype(jnp.bfloat16).astype(jnp.float32)
    qt = q.astype(jnp.bfloat16).astype(jnp.float32)
    d_all = (hid3t * qt[:, None, :]).sum(axis=-1)

    sel_logit = []
    sel_mask = []
    work = scores
    big = jnp.int32(2 * T)
    for _ in range(KW):
        mx = work.max(axis=1, keepdims=True)
        cand = jnp.where(work == mx, tpos, big)
        pick = cand.min(axis=1, keepdims=True)
        onehot = tpos == pick
        sel_mask.append(onehot)
        sel_logit.append(jnp.where(onehot, d_all, 0.0).sum(axis=1))
        work = jnp.where(onehot, neg, work)

    s = jnp.stack(sel_logit, axis=1)
    mmax = jnp.maximum(s.max(axis=1), 0.0)
    e = jnp.exp(s - mmax[:, None])
    z = e.sum(axis=1) + (S - KW) * jnp.exp(-mmax)
    a = e / z[:, None]

    w_t = jnp.zeros((_BT, T), jnp.float32)
    for kk in range(KW):
        w_t = w_t + jnp.where(sel_mask[kk], a[:, kk:kk + 1], 0.0)
    ctx_ref[...] = (w_t[:, :, None] * hid3).sum(axis=1)


def _encoder(h, w1, b1, w2, b2, ln_g, ln_b, wg, bg, wr, br):
    grid = B // _BT
    full = lambda shape: pl.BlockSpec(shape, lambda i: (0,) * len(shape))
    return pl.pallas_call(
        _encoder_body,
        grid=(grid,),
        in_specs=[
            pl.BlockSpec((_BT, T, D), lambda i: (i, 0, 0)),
            full((D, 2 * D)), full((2 * D,)),
            full((2 * D, D)), full((D,)),
            full((D,)), full((D,)),
            full((D, S)), full((S,)),
            full((D, D)), full((D,)),
        ],
        out_specs=pl.BlockSpec((_BT, D), lambda i: (i, 0)),
        out_shape=jax.ShapeDtypeStruct((B, D), jnp.float32),
    )(h, w1, b1, w2, b2, ln_g, ln_b, wg, bg, wr, br)


_VT = 2048


def _proj_body(ctx_ref, wo_ref, bo_ref, out_ref):
    out_ref[...] = jnp.dot(ctx_ref[...].astype(jnp.bfloat16),
                           wo_ref[...].astype(jnp.bfloat16),
                           preferred_element_type=jnp.float32) + bo_ref[...]


def _projection(ctx, wo, bo2d):
    grid = pl.cdiv(V, _VT)
    return pl.pallas_call(
        _proj_body,
        grid=(grid,),
        in_specs=[
            pl.BlockSpec((B, D), lambda j: (0, 0)),
            pl.BlockSpec((D, _VT), lambda j: (0, j)),
            pl.BlockSpec((1, _VT), lambda j: (0, j)),
        ],
        out_specs=pl.BlockSpec((B, _VT), lambda j: (0, j)),
        out_shape=jax.ShapeDtypeStruct((B, V), jnp.float32),
    )(ctx, wo, bo2d)


def kernel(seq, table, w1, b1, w2, b2, ln_g, ln_b, wg, bg, wr, br, wo, bo):
    idx3d = seq.reshape(32, -1, _CHUNK).astype(jnp.int32)
    h = _sc_gather(table, idx3d)
    ctx = _encoder(h.reshape(B, T, D), w1, b1, w2, b2,
                   ln_g, ln_b, wg, bg, wr, br)
    return _projection(ctx, wo, bo.reshape(1, V))
